# baseline (device time: 22729 ns/iter reference)
import math

import jax
import jax.numpy as jnp
from jax import lax
from jax.experimental import pallas as pl
from jax.experimental.pallas import tpu as pltpu

N_DEV = 4


def kernel(q, k, v):
    S, D = q.shape
    H = S // 2

    def body(q_ref, k_ref, v_ref, out_ref,
             snd_ref, nbrl_ref, nbrr_ref, diag_ref, send_sems, recv_sems):
        my = lax.axis_index("i")
        left = (my - 1) % N_DEV
        right = (my + 1) % N_DEV

        snd_ref[0] = k_ref[...].astype(jnp.bfloat16)
        snd_ref[1] = v_ref[...].astype(jnp.bfloat16)
        q_scaled = (
            q_ref[...] * (math.log2(math.e) / math.sqrt(D))
        ).astype(jnp.bfloat16)

        barrier_sem = pltpu.get_barrier_semaphore()
        for nbr in [left, right]:
            pl.semaphore_signal(
                barrier_sem, inc=1,
                device_id=(nbr,), device_id_type=pl.DeviceIdType.MESH,
            )
        pl.semaphore_wait(barrier_sem, 2)

        to_right = pltpu.make_async_remote_copy(
            src_ref=snd_ref, dst_ref=nbrl_ref,
            send_sem=send_sems.at[0], recv_sem=recv_sems.at[0],
            device_id=(right,), device_id_type=pl.DeviceIdType.MESH,
        )
        to_right.start()
        to_left = pltpu.make_async_remote_copy(
            src_ref=snd_ref, dst_ref=nbrr_ref,
            send_sem=send_sems.at[1], recv_sem=recv_sems.at[1],
            device_id=(left,), device_id_type=pl.DeviceIdType.MESH,
        )
        to_left.start()

        l = jnp.zeros((S, 1), dtype=jnp.float32)
        acc = jnp.zeros((S, D), dtype=jnp.float32)

        def absorb(l, acc, k_blk, v_blk):
            s = jax.lax.dot_general(
                q_scaled, k_blk,
                dimension_numbers=(((1,), (1,)), ((), ())),
                preferred_element_type=jnp.float32,
            )
            p = jnp.exp2(s.astype(jnp.bfloat16))
            l = l + jnp.sum(p, axis=1, keepdims=True, dtype=jnp.float32)
            acc = acc + jax.lax.dot_general(
                p, v_blk,
                dimension_numbers=(((1,), (0,)), ((), ())),
                preferred_element_type=jnp.float32,
            )
            return l, acc

        l, acc = absorb(l, acc, snd_ref[0], snd_ref[1])

        to_right.wait_recv()
        fwd_right = pltpu.make_async_remote_copy(
            src_ref=nbrl_ref.at[:, :H, :], dst_ref=diag_ref.at[:, :H, :],
            send_sem=send_sems.at[2], recv_sem=recv_sems.at[2],
            device_id=(right,), device_id_type=pl.DeviceIdType.MESH,
        )
        fwd_right.start()
        l, acc = absorb(l, acc, nbrl_ref[0], nbrl_ref[1])

        to_left.wait_recv()
        fwd_left = pltpu.make_async_remote_copy(
            src_ref=nbrr_ref.at[:, H:, :], dst_ref=diag_ref.at[:, H:, :],
            send_sem=send_sems.at[3], recv_sem=recv_sems.at[3],
            device_id=(left,), device_id_type=pl.DeviceIdType.MESH,
        )
        fwd_left.start()
        l, acc = absorb(l, acc, nbrr_ref[0], nbrr_ref[1])

        fwd_right.wait_recv()
        fwd_left.wait_recv()
        l, acc = absorb(l, acc, diag_ref[0], diag_ref[1])

        for r in (to_right, to_left, fwd_right, fwd_left):
            r.wait_send()

        out_ref[...] = acc / l

    return pl.pallas_call(
        body,
        out_shape=jax.ShapeDtypeStruct((S, D), jnp.float32),
        in_specs=[pl.BlockSpec(memory_space=pltpu.VMEM)] * 3,
        out_specs=pl.BlockSpec(memory_space=pltpu.VMEM),
        scratch_shapes=[
            pltpu.VMEM((2, S, D), jnp.bfloat16),
            pltpu.VMEM((2, S, D), jnp.bfloat16),
            pltpu.VMEM((2, S, D), jnp.bfloat16),
            pltpu.VMEM((2, S, D), jnp.bfloat16),
            pltpu.SemaphoreType.DMA((4,)),
            pltpu.SemaphoreType.DMA((4,)),
        ],
        compiler_params=pltpu.CompilerParams(collective_id=0),
    )(q, k, v)


# device time: 18172 ns/iter; 1.2508x vs baseline; 1.2508x over previous
import math

import jax
import jax.numpy as jnp
from jax import lax
from jax.experimental import pallas as pl
from jax.experimental.pallas import tpu as pltpu

N_DEV = 4


def kernel(q, k, v):
    S, D = q.shape
    H = S // 2

    def body(q_ref, k_ref, v_ref, out_ref,
             snd_ref, nbrl_ref, nbrr_ref, diag_ref, send_sems, recv_sems):
        my = lax.axis_index("i")
        left = (my - 1) % N_DEV
        right = (my + 1) % N_DEV

        snd_ref[0] = k_ref[...].astype(jnp.bfloat16)
        snd_ref[1] = v_ref[...].astype(jnp.bfloat16)
        q_scaled = (
            q_ref[...] * (math.log2(math.e) / math.sqrt(D))
        ).astype(jnp.bfloat16)

        barrier_sem = pltpu.get_barrier_semaphore()
        for nbr in [left, right]:
            pl.semaphore_signal(
                barrier_sem, inc=1,
                device_id=(nbr,), device_id_type=pl.DeviceIdType.MESH,
            )
        pl.semaphore_wait(barrier_sem, 2)

        def copy(src, dst, i, dev):
            rdma = pltpu.make_async_remote_copy(
                src_ref=src, dst_ref=dst,
                send_sem=send_sems.at[i], recv_sem=recv_sems.at[i],
                device_id=(dev,), device_id_type=pl.DeviceIdType.MESH,
            )
            rdma.start()
            return rdma

        lo = slice(None, H)
        hi = slice(H, None)

        dr_lo = copy(snd_ref.at[:, lo, :], nbrl_ref.at[:, lo, :], 0, right)
        dr_hi = copy(snd_ref.at[:, hi, :], nbrl_ref.at[:, hi, :], 1, right)
        dl_hi = copy(snd_ref.at[:, hi, :], nbrr_ref.at[:, hi, :], 2, left)
        dl_lo = copy(snd_ref.at[:, lo, :], nbrr_ref.at[:, lo, :], 3, left)

        l = jnp.zeros((S, 1), dtype=jnp.float32)
        acc = jnp.zeros((S, D), dtype=jnp.float32)

        def absorb(l, acc, k_blk, v_blk):
            s = jax.lax.dot_general(
                q_scaled, k_blk,
                dimension_numbers=(((1,), (1,)), ((), ())),
                preferred_element_type=jnp.float32,
            )
            p = jnp.exp2(s.astype(jnp.bfloat16))
            l = l + jnp.sum(p, axis=1, keepdims=True, dtype=jnp.float32)
            acc = acc + jax.lax.dot_general(
                p, v_blk,
                dimension_numbers=(((1,), (0,)), ((), ())),
                preferred_element_type=jnp.float32,
            )
            return l, acc

        l, acc = absorb(l, acc, k_ref[lo, :].astype(jnp.bfloat16),
                        v_ref[lo, :].astype(jnp.bfloat16))
        l, acc = absorb(l, acc, k_ref[hi, :].astype(jnp.bfloat16),
                        v_ref[hi, :].astype(jnp.bfloat16))

        dr_lo.wait_recv()
        relay_r = copy(nbrl_ref.at[:, lo, :], diag_ref.at[:, lo, :], 4, right)
        l, acc = absorb(l, acc, nbrl_ref[0, lo, :], nbrl_ref[1, lo, :])

        dl_hi.wait_recv()
        relay_l = copy(nbrr_ref.at[:, hi, :], diag_ref.at[:, hi, :], 5, left)
        l, acc = absorb(l, acc, nbrr_ref[0, hi, :], nbrr_ref[1, hi, :])

        dr_hi.wait_recv()
        l, acc = absorb(l, acc, nbrl_ref[0, hi, :], nbrl_ref[1, hi, :])
        dl_lo.wait_recv()
        l, acc = absorb(l, acc, nbrr_ref[0, lo, :], nbrr_ref[1, lo, :])

        relay_r.wait_recv()
        l, acc = absorb(l, acc, diag_ref[0, lo, :], diag_ref[1, lo, :])
        relay_l.wait_recv()
        l, acc = absorb(l, acc, diag_ref[0, hi, :], diag_ref[1, hi, :])

        for r in (dr_lo, dr_hi, dl_hi, dl_lo, relay_r, relay_l):
            r.wait_send()

        out_ref[...] = acc / l

    return pl.pallas_call(
        body,
        out_shape=jax.ShapeDtypeStruct((S, D), jnp.float32),
        in_specs=[pl.BlockSpec(memory_space=pltpu.VMEM)] * 3,
        out_specs=pl.BlockSpec(memory_space=pltpu.VMEM),
        scratch_shapes=[
            pltpu.VMEM((2, S, D), jnp.bfloat16),
            pltpu.VMEM((2, S, D), jnp.bfloat16),
            pltpu.VMEM((2, S, D), jnp.bfloat16),
            pltpu.VMEM((2, S, D), jnp.bfloat16),
            pltpu.SemaphoreType.DMA((6,)),
            pltpu.SemaphoreType.DMA((6,)),
        ],
        compiler_params=pltpu.CompilerParams(collective_id=0),
    )(q, k, v)
